# Initial kernel scaffold; baseline (speedup 1.0000x reference)
import jax
import jax.numpy as jnp
from jax.experimental import pallas as pl


def kernel(grad_output, index):
    raise NotImplementedError("write your pallas kernel here")


# speedup vs baseline: 1.5472x; 1.5472x over previous
import functools

import jax
import jax.numpy as jnp
from jax import lax
from jax.experimental import pallas as pl
from jax.experimental.pallas import tpu as pltpu
from jax.experimental.pallas import tpu_sc as plsc

NUM_WEIGHTS = 512
PADDING_IDX = 1
LANES = 16
NUM_WORKERS = 32


def _build(T, D, V):
    R = V // NUM_WORKERS
    mesh = plsc.VectorSubcoreMesh(core_axis_name="c", subcore_axis_name="s")

    @functools.partial(
        pl.kernel,
        mesh=mesh,
        out_type=jax.ShapeDtypeStruct((V * D,), jnp.float32),
        scratch_types=[
            pltpu.VMEM((T,), jnp.int32),
            pltpu.VMEM((D,), jnp.float32),
            pltpu.VMEM((R * D,), jnp.float32),
        ],
    )
    def k(grad_hbm, idx_hbm, out_hbm, idx_v, row_v, acc_v):
        wid = lax.axis_index("s") * 2 + lax.axis_index("c")
        base = wid * R

        pltpu.sync_copy(idx_hbm, idx_v)

        def zero_body(i, _):
            acc_v[pl.ds(i * LANES, LANES)] = jnp.zeros((LANES,), jnp.float32)
            return 0
        lax.fori_loop(0, R * D // LANES, zero_body, 0)

        row_ids = base + lax.broadcasted_iota(jnp.int32, (LANES,), 0)

        def grp_body(g, cnt16):
            rvec = idx_v[pl.ds(g * LANES, LANES)]
            for lane in range(LANES):
                r = rvec[lane]
                t = g * LANES + lane
                cnt16 = cnt16 + jnp.where(row_ids == r, 1.0, 0.0)
                matched = (r >= base) & (r < base + R) & (r != PADDING_IDX)

                @pl.when(matched)
                def _(r=r, t=t):
                    pltpu.sync_copy(grad_hbm.at[pl.ds(t * D, D)], row_v)
                    loc = (r - base) * D
                    for j in range(D // LANES):
                        sl = pl.ds(loc + j * LANES, LANES)
                        acc_v[sl] = acc_v[sl] + row_v[pl.ds(j * LANES, LANES)]
            return cnt16
        cnt16 = lax.fori_loop(0, T // LANES, grp_body,
                              jnp.zeros((LANES,), jnp.float32))

        for i in range(R):
            c = jnp.maximum(cnt16[i], 1.0)

            def div_body(j, _, i=i, c=c):
                sl = pl.ds(i * D + j * LANES, LANES)
                acc_v[sl] = acc_v[sl] / c
                return 0
            lax.fori_loop(0, D // LANES, div_body, 0)

        pltpu.sync_copy(acc_v, out_hbm.at[pl.ds(base * D, R * D)])

    return k


def kernel(grad_output, index):
    T = index.shape[0] * index.shape[1]
    D = grad_output.shape[-1]
    go = grad_output.reshape(-1).astype(jnp.float32)
    idx = index.reshape(-1).astype(jnp.int32)
    out = _build(T, D, NUM_WEIGHTS)(go, idx)
    return out.reshape(NUM_WEIGHTS, D)


# speedup vs baseline: 1.6372x; 1.0582x over previous
import functools

import jax
import jax.numpy as jnp
from jax import lax
from jax.experimental import pallas as pl
from jax.experimental.pallas import tpu as pltpu
from jax.experimental.pallas import tpu_sc as plsc

NUM_WEIGHTS = 512
PADDING_IDX = 1
LANES = 16
NUM_WORKERS = 32


def _build(T, D, V):
    R = V // NUM_WORKERS
    mesh = plsc.VectorSubcoreMesh(core_axis_name="c", subcore_axis_name="s")

    @functools.partial(
        pl.kernel,
        mesh=mesh,
        out_type=jax.ShapeDtypeStruct((V * D,), jnp.float32),
        scratch_types=[
            pltpu.VMEM((T,), jnp.int32),
            pltpu.VMEM((D,), jnp.float32),
            pltpu.VMEM((R * D,), jnp.float32),
        ],
    )
    def k(grad_hbm, idx_hbm, out_hbm, idx_v, row_v, acc_v):
        wid = lax.axis_index("s") * 2 + lax.axis_index("c")
        base = wid * R

        pltpu.sync_copy(idx_hbm, idx_v)

        def zero_body(i, _):
            acc_v[pl.ds(i * LANES, LANES)] = jnp.zeros((LANES,), jnp.float32)
            return 0
        lax.fori_loop(0, R * D // LANES, zero_body, 0, unroll=8)

        row_ids = base + lax.broadcasted_iota(jnp.int32, (LANES,), 0)

        def grp_body(g, cnt16):
            rvec = idx_v[pl.ds(g * LANES, LANES)]
            for lane in range(LANES):
                r = rvec[lane]
                t = g * LANES + lane
                cnt16 = cnt16 + jnp.where(row_ids == r, 1.0, 0.0)
                matched = (r >= base) & (r < base + R) & (r != PADDING_IDX)

                @pl.when(matched)
                def _(r=r, t=t):
                    pltpu.sync_copy(grad_hbm.at[pl.ds(t * D, D)], row_v)
                    loc = (r - base) * D
                    for j in range(D // LANES):
                        sl = pl.ds(loc + j * LANES, LANES)
                        acc_v[sl] = acc_v[sl] + row_v[pl.ds(j * LANES, LANES)]
            return cnt16
        cnt16 = lax.fori_loop(0, T // LANES, grp_body,
                              jnp.zeros((LANES,), jnp.float32))

        inv16 = 1.0 / jnp.maximum(cnt16, 1.0)
        for i in range(R):
            @pl.when(cnt16[i] > 1.0)
            def _(i=i):
                iv = inv16[i]

                def sc_body(j, _, i=i, iv=iv):
                    sl = pl.ds(i * D + j * LANES, LANES)
                    acc_v[sl] = acc_v[sl] * iv
                    return 0
                lax.fori_loop(0, D // LANES, sc_body, 0, unroll=6)

        pltpu.sync_copy(acc_v, out_hbm.at[pl.ds(base * D, R * D)])

    return k


def kernel(grad_output, index):
    T = index.shape[0] * index.shape[1]
    D = grad_output.shape[-1]
    go = grad_output.reshape(-1).astype(jnp.float32)
    idx = index.reshape(-1).astype(jnp.int32)
    out = _build(T, D, NUM_WEIGHTS)(go, idx)
    return out.reshape(NUM_WEIGHTS, D)
